# baseline (speedup 1.0000x reference)
import functools

import jax
import jax.numpy as jnp
from jax import lax
from jax.experimental import pallas as pl
from jax.experimental.pallas import tpu as pltpu
from jax.experimental.pallas import tpu_sc as plsc

_B, _T, _D = 4, 8192, 2048
_H1, _E = 256, 64

_NC, _NS = 2, 16
_NW = _NC * _NS
_RPT = _T // _NW
_RC = 16
_NCH = _RPT // _RC
_NCHT = _B * _NCH
_CW = _D // 16


def _sc_body(x_hbm, out_hbm, buf0, buf1, acc, sem0, sem1):
    wid = lax.axis_index("c") * _NS + lax.axis_index("s")
    row0 = wid * _RPT

    def _zero(i, carry):
        acc[i // _CW, pl.ds((i % _CW) * 16, 16)] = jnp.zeros((16,), jnp.float32)
        return carry

    lax.fori_loop(0, _B * _CW, _zero, 0)

    def _src(j):
        b = j // _NCH
        i = j - b * _NCH
        return x_hbm.at[b, pl.ds(row0 + i * _RC, _RC), :]

    def _consume(j, buf):
        b = j // _NCH

        def cbody(c, carry):
            o = c * 16
            s = buf[0, pl.ds(o, 16)]
            for r in range(1, _RC):
                s = s + buf[r, pl.ds(o, 16)]
            plsc.addupdate(acc.at[b, pl.ds(o, 16)], s)
            return carry

        lax.fori_loop(0, _CW, cbody, 0)

    def _pbody(jj, carry):
        j0 = jj * 2
        j1 = j0 + 1
        pltpu.make_async_copy(_src(j0), buf0, sem0).wait()
        _consume(j0, buf0)

        @pl.when(j0 + 2 < _NCHT)
        def _():
            pltpu.async_copy(_src(j0 + 2), buf0, sem0)

        pltpu.make_async_copy(_src(j1), buf1, sem1).wait()
        _consume(j1, buf1)

        @pl.when(j1 + 2 < _NCHT)
        def _():
            pltpu.async_copy(_src(j1 + 2), buf1, sem1)

        return carry

    pltpu.async_copy(_src(0), buf0, sem0)
    pltpu.async_copy(_src(1), buf1, sem1)
    lax.fori_loop(0, _NCHT // 2, _pbody, 0)
    pltpu.sync_copy(acc, out_hbm.at[wid])


_sc_reduce = functools.partial(
    pl.kernel,
    out_type=jax.ShapeDtypeStruct((_NW, _B, _D), jnp.float32),
    mesh=plsc.VectorSubcoreMesh(core_axis_name="c", subcore_axis_name="s"),
    scratch_types=[
        pltpu.VMEM((_RC, _D), jnp.float32),
        pltpu.VMEM((_RC, _D), jnp.float32),
        pltpu.VMEM((_B, _D), jnp.float32),
        pltpu.SemaphoreType.DMA,
        pltpu.SemaphoreType.DMA,
    ],
)(_sc_body)


def _gate_body(p_ref, w1_ref, b1_ref, w2_ref, b2_ref, w_ref, i_ref):
    xm = jnp.sum(p_ref[...], axis=0) * (1.0 / _T)
    h = jnp.maximum(
        jnp.dot(xm, w1_ref[...], preferred_element_type=jnp.float32)
        + b1_ref[...], 0.0)
    g = (jnp.dot(h, w2_ref[...], preferred_element_type=jnp.float32)
         + b2_ref[...])
    gmax = jnp.max(g, axis=-1, keepdims=True)
    e = jnp.exp(g - gmax)
    p = e / jnp.sum(e, axis=-1, keepdims=True)
    iota = lax.broadcasted_iota(jnp.int32, p.shape, 1)
    v1 = jnp.max(p, axis=-1, keepdims=True)
    i1 = jnp.min(jnp.where(p == v1, iota, _E), axis=-1, keepdims=True)
    p2 = jnp.where(iota == i1, -jnp.inf, p)
    v2 = jnp.max(p2, axis=-1, keepdims=True)
    i2 = jnp.min(jnp.where(p2 == v2, iota, _E), axis=-1, keepdims=True)
    s = v1 + v2
    w_ref[...] = jnp.concatenate([v1 / s, v2 / s], axis=1)
    i_ref[...] = jnp.concatenate([i1, i2], axis=1)


@jax.jit
def _run(x, W1, b1, W2, b2):
    partials = _sc_reduce(x)
    w, idx = pl.pallas_call(
        _gate_body,
        out_shape=[
            jax.ShapeDtypeStruct((_B, 2), jnp.float32),
            jax.ShapeDtypeStruct((_B, 2), jnp.int32),
        ],
    )(partials, W1, b1.reshape(1, _H1), W2, b2.reshape(1, _E))
    return w, idx


def kernel(x, W1, b1, W2, b2):
    return _run(x, W1, b1, W2, b2)


# speedup vs baseline: 1.4868x; 1.4868x over previous
import functools

import jax
import jax.numpy as jnp
from jax import lax
from jax.experimental import pallas as pl
from jax.experimental.pallas import tpu as pltpu
from jax.experimental.pallas import tpu_sc as plsc

_B, _T, _D = 4, 8192, 2048
_H1, _E = 256, 64

_NC, _NS = 2, 16
_NW = _NC * _NS
_RPT = _T // _NW
_RC = 16
_NCH = _RPT // _RC
_NCHT = _B * _NCH
_CW = _D // 16


def _sc_body(x_hbm, out_hbm, buf0, buf1, acc, sem0, sem1):
    wid = lax.axis_index("c") * _NS + lax.axis_index("s")
    row0 = wid * _RPT

    def _zero(i, carry):
        acc[i // _CW, pl.ds((i % _CW) * 16, 16)] = jnp.zeros((16,), jnp.float32)
        return carry

    lax.fori_loop(0, _B * _CW, _zero, 0)

    def _src(j):
        b = j // _NCH
        i = j - b * _NCH
        return x_hbm.at[b, pl.ds(row0 + i * _RC, _RC), :]

    def _consume(j, buf):
        b = j // _NCH

        @plsc.parallel_loop(0, _CW, step=1, unroll=4)
        def cbody(c):
            o = c * 16
            s = buf[0, pl.ds(o, 16)]
            for r in range(1, _RC):
                s = s + buf[r, pl.ds(o, 16)]
            plsc.addupdate(acc.at[b, pl.ds(o, 16)], s)

    def _pbody(jj, carry):
        j0 = jj * 2
        j1 = j0 + 1
        pltpu.make_async_copy(_src(j0), buf0, sem0).wait()
        _consume(j0, buf0)

        @pl.when(j0 + 2 < _NCHT)
        def _():
            pltpu.async_copy(_src(j0 + 2), buf0, sem0)

        pltpu.make_async_copy(_src(j1), buf1, sem1).wait()
        _consume(j1, buf1)

        @pl.when(j1 + 2 < _NCHT)
        def _():
            pltpu.async_copy(_src(j1 + 2), buf1, sem1)

        return carry

    pltpu.async_copy(_src(0), buf0, sem0)
    pltpu.async_copy(_src(1), buf1, sem1)
    lax.fori_loop(0, _NCHT // 2, _pbody, 0)
    pltpu.sync_copy(acc, out_hbm.at[wid])


_sc_reduce = functools.partial(
    pl.kernel,
    out_type=jax.ShapeDtypeStruct((_NW, _B, _D), jnp.float32),
    mesh=plsc.VectorSubcoreMesh(core_axis_name="c", subcore_axis_name="s"),
    scratch_types=[
        pltpu.VMEM((_RC, _D), jnp.float32),
        pltpu.VMEM((_RC, _D), jnp.float32),
        pltpu.VMEM((_B, _D), jnp.float32),
        pltpu.SemaphoreType.DMA,
        pltpu.SemaphoreType.DMA,
    ],
)(_sc_body)


def _gate_body(p_ref, w1_ref, b1_ref, w2_ref, b2_ref, w_ref, i_ref):
    xm = jnp.sum(p_ref[...], axis=0) * (1.0 / _T)
    h = jnp.maximum(
        jnp.dot(xm, w1_ref[...], preferred_element_type=jnp.float32)
        + b1_ref[...], 0.0)
    g = (jnp.dot(h, w2_ref[...], preferred_element_type=jnp.float32)
         + b2_ref[...])
    gmax = jnp.max(g, axis=-1, keepdims=True)
    e = jnp.exp(g - gmax)
    p = e / jnp.sum(e, axis=-1, keepdims=True)
    iota = lax.broadcasted_iota(jnp.int32, p.shape, 1)
    v1 = jnp.max(p, axis=-1, keepdims=True)
    i1 = jnp.min(jnp.where(p == v1, iota, _E), axis=-1, keepdims=True)
    p2 = jnp.where(iota == i1, -jnp.inf, p)
    v2 = jnp.max(p2, axis=-1, keepdims=True)
    i2 = jnp.min(jnp.where(p2 == v2, iota, _E), axis=-1, keepdims=True)
    s = v1 + v2
    w_ref[...] = jnp.concatenate([v1 / s, v2 / s], axis=1)
    i_ref[...] = jnp.concatenate([i1, i2], axis=1)


@jax.jit
def _run(x, W1, b1, W2, b2):
    partials = _sc_reduce(x)
    w, idx = pl.pallas_call(
        _gate_body,
        out_shape=[
            jax.ShapeDtypeStruct((_B, 2), jnp.float32),
            jax.ShapeDtypeStruct((_B, 2), jnp.int32),
        ],
    )(partials, W1, b1.reshape(1, _H1), W2, b2.reshape(1, _E))
    return w, idx


def kernel(x, W1, b1, W2, b2):
    return _run(x, W1, b1, W2, b2)


# speedup vs baseline: 1.4897x; 1.0019x over previous
import functools

import jax
import jax.numpy as jnp
from jax import lax
from jax.experimental import pallas as pl
from jax.experimental.pallas import tpu as pltpu
from jax.experimental.pallas import tpu_sc as plsc

_B, _T, _D = 4, 8192, 2048
_H1, _E = 256, 64

_NC, _NS = 2, 16
_NW = _NC * _NS
_RPT = _T // _NW
_RC = 16
_NCH = _RPT // _RC
_NCHT = _B * _NCH
_CW = _D // 16


def _sc_body(x_hbm, out_hbm, buf0, buf1, acc, sem0, sem1):
    wid = lax.axis_index("c") * _NS + lax.axis_index("s")
    row0 = wid * _RPT

    def _zero(i, carry):
        acc[i // _CW, pl.ds((i % _CW) * 16, 16)] = jnp.zeros((16,), jnp.float32)
        return carry

    lax.fori_loop(0, _B * _CW, _zero, 0)

    def _src(j):
        b = j // _NCH
        i = j - b * _NCH
        return x_hbm.at[b, pl.ds(row0 + i * _RC, _RC), :]

    def _consume(j, buf):
        b = j // _NCH

        @plsc.parallel_loop(0, _CW, step=1, unroll=8)
        def cbody(c):
            o = c * 16
            s = buf[0, pl.ds(o, 16)]
            for r in range(1, _RC):
                s = s + buf[r, pl.ds(o, 16)]
            plsc.addupdate(acc.at[b, pl.ds(o, 16)], s)

    def _pbody(jj, carry):
        j0 = jj * 2
        j1 = j0 + 1
        pltpu.make_async_copy(_src(j0), buf0, sem0).wait()
        _consume(j0, buf0)

        @pl.when(j0 + 2 < _NCHT)
        def _():
            pltpu.async_copy(_src(j0 + 2), buf0, sem0)

        pltpu.make_async_copy(_src(j1), buf1, sem1).wait()
        _consume(j1, buf1)

        @pl.when(j1 + 2 < _NCHT)
        def _():
            pltpu.async_copy(_src(j1 + 2), buf1, sem1)

        return carry

    pltpu.async_copy(_src(0), buf0, sem0)
    pltpu.async_copy(_src(1), buf1, sem1)
    lax.fori_loop(0, _NCHT // 2, _pbody, 0)
    pltpu.sync_copy(acc, out_hbm.at[wid])


_sc_reduce = functools.partial(
    pl.kernel,
    out_type=jax.ShapeDtypeStruct((_NW, _B, _D), jnp.float32),
    mesh=plsc.VectorSubcoreMesh(core_axis_name="c", subcore_axis_name="s"),
    scratch_types=[
        pltpu.VMEM((_RC, _D), jnp.float32),
        pltpu.VMEM((_RC, _D), jnp.float32),
        pltpu.VMEM((_B, _D), jnp.float32),
        pltpu.SemaphoreType.DMA,
        pltpu.SemaphoreType.DMA,
    ],
)(_sc_body)


def _gate_body(p_ref, w1_ref, b1_ref, w2_ref, b2_ref, w_ref, i_ref):
    xm = jnp.sum(p_ref[...], axis=0) * (1.0 / _T)
    h = jnp.maximum(
        jnp.dot(xm, w1_ref[...], preferred_element_type=jnp.float32)
        + b1_ref[...], 0.0)
    g = (jnp.dot(h, w2_ref[...], preferred_element_type=jnp.float32)
         + b2_ref[...])
    gmax = jnp.max(g, axis=-1, keepdims=True)
    e = jnp.exp(g - gmax)
    p = e / jnp.sum(e, axis=-1, keepdims=True)
    iota = lax.broadcasted_iota(jnp.int32, p.shape, 1)
    v1 = jnp.max(p, axis=-1, keepdims=True)
    i1 = jnp.min(jnp.where(p == v1, iota, _E), axis=-1, keepdims=True)
    p2 = jnp.where(iota == i1, -jnp.inf, p)
    v2 = jnp.max(p2, axis=-1, keepdims=True)
    i2 = jnp.min(jnp.where(p2 == v2, iota, _E), axis=-1, keepdims=True)
    s = v1 + v2
    w_ref[...] = jnp.concatenate([v1 / s, v2 / s], axis=1)
    i_ref[...] = jnp.concatenate([i1, i2], axis=1)


@jax.jit
def _run(x, W1, b1, W2, b2):
    partials = _sc_reduce(x)
    w, idx = pl.pallas_call(
        _gate_body,
        out_shape=[
            jax.ShapeDtypeStruct((_B, 2), jnp.float32),
            jax.ShapeDtypeStruct((_B, 2), jnp.int32),
        ],
    )(partials, W1, b1.reshape(1, _H1), W2, b2.reshape(1, _E))
    return w, idx


def kernel(x, W1, b1, W2, b2):
    return _run(x, W1, b1, W2, b2)


# speedup vs baseline: 1.9240x; 1.2915x over previous
import functools

import jax
import jax.numpy as jnp
from jax import lax
from jax.experimental import pallas as pl
from jax.experimental.pallas import tpu as pltpu
from jax.experimental.pallas import tpu_sc as plsc

_B, _T, _D = 4, 8192, 2048
_H1, _E = 256, 64

_TSC = 3072
_NC, _NS = 2, 16
_NW = _NC * _NS
_RPT = _TSC // _NW
_RC = 16
_NCH = _RPT // _RC
_NCHT = _B * _NCH
_CW = _D // 16

_CHUNK = 512
_KTC = (_T - _TSC) // _CHUNK
_BLK0 = _TSC // _CHUNK


def _sc_body(x_hbm, out_hbm, buf0, buf1, acc, sem0, sem1):
    wid = lax.axis_index("c") * _NS + lax.axis_index("s")
    row0 = wid * _RPT

    def _zero(i, carry):
        acc[i // _CW, pl.ds((i % _CW) * 16, 16)] = jnp.zeros((16,), jnp.float32)
        return carry

    lax.fori_loop(0, _B * _CW, _zero, 0)

    def _src(j):
        b = j // _NCH
        i = j - b * _NCH
        return x_hbm.at[b, pl.ds(row0 + i * _RC, _RC), :]

    def _consume(j, buf):
        b = j // _NCH

        @plsc.parallel_loop(0, _CW, step=1, unroll=8)
        def cbody(c):
            o = c * 16
            s = buf[0, pl.ds(o, 16)]
            for r in range(1, _RC):
                s = s + buf[r, pl.ds(o, 16)]
            plsc.addupdate(acc.at[b, pl.ds(o, 16)], s)

    def _pbody(jj, carry):
        j0 = jj * 2
        j1 = j0 + 1
        pltpu.make_async_copy(_src(j0), buf0, sem0).wait()
        _consume(j0, buf0)

        @pl.when(j0 + 2 < _NCHT)
        def _():
            pltpu.async_copy(_src(j0 + 2), buf0, sem0)

        pltpu.make_async_copy(_src(j1), buf1, sem1).wait()
        _consume(j1, buf1)

        @pl.when(j1 + 2 < _NCHT)
        def _():
            pltpu.async_copy(_src(j1 + 2), buf1, sem1)

        return carry

    pltpu.async_copy(_src(0), buf0, sem0)
    pltpu.async_copy(_src(1), buf1, sem1)
    lax.fori_loop(0, _NCHT // 2, _pbody, 0)
    pltpu.sync_copy(acc, out_hbm.at[wid])


_sc_reduce = functools.partial(
    pl.kernel,
    out_type=jax.ShapeDtypeStruct((_NW, _B, _D), jnp.float32),
    mesh=plsc.VectorSubcoreMesh(core_axis_name="c", subcore_axis_name="s"),
    scratch_types=[
        pltpu.VMEM((_RC, _D), jnp.float32),
        pltpu.VMEM((_RC, _D), jnp.float32),
        pltpu.VMEM((_B, _D), jnp.float32),
        pltpu.SemaphoreType.DMA,
        pltpu.SemaphoreType.DMA,
    ],
)(_sc_body)


def _tc_body(x_ref, o_ref):
    k = pl.program_id(0)
    part = jnp.sum(x_ref[...], axis=1)

    @pl.when(k == 0)
    def _init():
        o_ref[:, 0, :] = part

    @pl.when(k > 0)
    def _acc():
        o_ref[:, 0, :] += part


def _gate_body(sc_ref, tc_ref, w1_ref, b1_ref, w2_ref, b2_ref, w_ref, i_ref):
    total = jnp.sum(sc_ref[...], axis=0) + tc_ref[:, 0, :]
    xm = total * (1.0 / _T)
    h = jnp.maximum(
        jnp.dot(xm, w1_ref[...], preferred_element_type=jnp.float32)
        + b1_ref[...], 0.0)
    g = (jnp.dot(h, w2_ref[...], preferred_element_type=jnp.float32)
         + b2_ref[...])
    gmax = jnp.max(g, axis=-1, keepdims=True)
    e = jnp.exp(g - gmax)
    p = e / jnp.sum(e, axis=-1, keepdims=True)
    iota = lax.broadcasted_iota(jnp.int32, p.shape, 1)
    v1 = jnp.max(p, axis=-1, keepdims=True)
    i1 = jnp.min(jnp.where(p == v1, iota, _E), axis=-1, keepdims=True)
    p2 = jnp.where(iota == i1, -jnp.inf, p)
    v2 = jnp.max(p2, axis=-1, keepdims=True)
    i2 = jnp.min(jnp.where(p2 == v2, iota, _E), axis=-1, keepdims=True)
    s = v1 + v2
    w_ref[...] = jnp.concatenate([v1 / s, v2 / s], axis=1)
    i_ref[...] = jnp.concatenate([i1, i2], axis=1)


@jax.jit
def _run(x, W1, b1, W2, b2):
    sc_partials = _sc_reduce(x)
    tc_partial = pl.pallas_call(
        _tc_body,
        grid=(_KTC,),
        in_specs=[pl.BlockSpec((_B, _CHUNK, _D), lambda k: (0, _BLK0 + k, 0))],
        out_specs=pl.BlockSpec((_B, 1, _D), lambda k: (0, 0, 0)),
        out_shape=jax.ShapeDtypeStruct((_B, 1, _D), jnp.float32),
        compiler_params=pltpu.CompilerParams(
            dimension_semantics=("arbitrary",)),
    )(x)
    w, idx = pl.pallas_call(
        _gate_body,
        out_shape=[
            jax.ShapeDtypeStruct((_B, 2), jnp.float32),
            jax.ShapeDtypeStruct((_B, 2), jnp.int32),
        ],
    )(sc_partials, tc_partial, W1, b1.reshape(1, _H1), W2,
      b2.reshape(1, _E))
    return w, idx


def kernel(x, W1, b1, W2, b2):
    return _run(x, W1, b1, W2, b2)


# speedup vs baseline: 2.3784x; 1.2362x over previous
import functools

import jax
import jax.numpy as jnp
from jax import lax
from jax.experimental import pallas as pl
from jax.experimental.pallas import tpu as pltpu

_B, _T, _D = 4, 8192, 2048
_H1, _E = 256, 64
_CHUNK = 256
_K = _T // _CHUNK
_R = _B * _CHUNK


def _body(x_ref, w1_ref, b1_ref, w2_ref, b2_ref, w_ref, i_ref, acc_ref):
    k = pl.program_id(0)

    row = lax.broadcasted_iota(jnp.int32, (8, _R), 0)
    col = lax.broadcasted_iota(jnp.int32, (8, _R), 1)
    sel = (col // _CHUNK == row % _B).astype(jnp.float32)
    part = jnp.dot(sel, x_ref[...].reshape(_R, _D),
                   preferred_element_type=jnp.float32)

    @pl.when(k == 0)
    def _init():
        acc_ref[...] = part

    @pl.when(k > 0)
    def _acc():
        acc_ref[...] += part

    @pl.when(k == _K - 1)
    def _gate():
        xm = acc_ref[0:_B, :] * (1.0 / _T)
        h = jnp.maximum(
            jnp.dot(xm, w1_ref[...], preferred_element_type=jnp.float32)
            + b1_ref[...], 0.0)
        g = (jnp.dot(h, w2_ref[...], preferred_element_type=jnp.float32)
             + b2_ref[...])
        gmax = jnp.max(g, axis=-1, keepdims=True)
        e = jnp.exp(g - gmax)
        p = e / jnp.sum(e, axis=-1, keepdims=True)
        iota = lax.broadcasted_iota(jnp.int32, p.shape, 1)
        v1 = jnp.max(p, axis=-1, keepdims=True)
        i1 = jnp.min(jnp.where(p == v1, iota, _E), axis=-1, keepdims=True)
        p2 = jnp.where(iota == i1, -jnp.inf, p)
        v2 = jnp.max(p2, axis=-1, keepdims=True)
        i2 = jnp.min(jnp.where(p2 == v2, iota, _E), axis=-1, keepdims=True)
        s = v1 + v2
        w_ref[...] = jnp.concatenate([v1 / s, v2 / s], axis=1)
        i_ref[...] = jnp.concatenate([i1, i2], axis=1)


@functools.partial(jax.jit, static_argnames=("interpret",))
def _run(x, W1, b1, W2, b2, interpret=False):
    w, idx = pl.pallas_call(
        _body,
        grid=(_K,),
        in_specs=[
            pl.BlockSpec((_B, _CHUNK, _D), lambda k: (0, k, 0)),
            pl.BlockSpec((_D, _H1), lambda k: (0, 0)),
            pl.BlockSpec((1, _H1), lambda k: (0, 0)),
            pl.BlockSpec((_H1, _E), lambda k: (0, 0)),
            pl.BlockSpec((1, _E), lambda k: (0, 0)),
        ],
        out_specs=[
            pl.BlockSpec((_B, 2), lambda k: (0, 0)),
            pl.BlockSpec((_B, 2), lambda k: (0, 0)),
        ],
        out_shape=[
            jax.ShapeDtypeStruct((_B, 2), jnp.float32),
            jax.ShapeDtypeStruct((_B, 2), jnp.int32),
        ],
        scratch_shapes=[pltpu.VMEM((8, _D), jnp.float32)],
        compiler_params=pltpu.CompilerParams(
            dimension_semantics=("arbitrary",)),
        interpret=interpret,
    )(x, W1, b1.reshape(1, _H1), W2, b2.reshape(1, _E))
    return w, idx


def kernel(x, W1, b1, W2, b2):
    return _run(x, W1, b1, W2, b2)


# speedup vs baseline: 2.3822x; 1.0016x over previous
import functools

import jax
import jax.numpy as jnp
from jax import lax
from jax.experimental import pallas as pl
from jax.experimental.pallas import tpu as pltpu

_B, _T, _D = 4, 8192, 2048
_H1, _E = 256, 64
_ROWS = 1024
_K = (_B * _T) // _ROWS
_KPB = _T // _ROWS


def _body(x_ref, w1_ref, b1_ref, w2_ref, b2_ref, w_ref, i_ref, acc_ref):
    k = pl.program_id(0)
    b = k // _KPB

    row = lax.broadcasted_iota(jnp.int32, (8, _ROWS), 0)
    sel = (row % _B == b).astype(jnp.float32)
    part = jnp.dot(sel, x_ref[...], preferred_element_type=jnp.float32)

    @pl.when(k == 0)
    def _init():
        acc_ref[...] = part

    @pl.when(k > 0)
    def _acc():
        acc_ref[...] += part

    @pl.when(k == _K - 1)
    def _gate():
        xm = acc_ref[0:_B, :] * (1.0 / _T)
        h = jnp.maximum(
            jnp.dot(xm, w1_ref[...], preferred_element_type=jnp.float32)
            + b1_ref[...], 0.0)
        g = (jnp.dot(h, w2_ref[...], preferred_element_type=jnp.float32)
             + b2_ref[...])
        gmax = jnp.max(g, axis=-1, keepdims=True)
        e = jnp.exp(g - gmax)
        p = e / jnp.sum(e, axis=-1, keepdims=True)
        iota = lax.broadcasted_iota(jnp.int32, p.shape, 1)
        v1 = jnp.max(p, axis=-1, keepdims=True)
        i1 = jnp.min(jnp.where(p == v1, iota, _E), axis=-1, keepdims=True)
        p2 = jnp.where(iota == i1, -jnp.inf, p)
        v2 = jnp.max(p2, axis=-1, keepdims=True)
        i2 = jnp.min(jnp.where(p2 == v2, iota, _E), axis=-1, keepdims=True)
        s = v1 + v2
        w_ref[...] = jnp.concatenate([v1 / s, v2 / s], axis=1)
        i_ref[...] = jnp.concatenate([i1, i2], axis=1)


@functools.partial(jax.jit, static_argnames=("interpret",))
def _run(x, W1, b1, W2, b2, interpret=False):
    x2 = x.reshape(_B * _T, _D)
    w, idx = pl.pallas_call(
        _body,
        grid=(_K,),
        in_specs=[
            pl.BlockSpec((_ROWS, _D), lambda k: (k, 0)),
            pl.BlockSpec((_D, _H1), lambda k: (0, 0)),
            pl.BlockSpec((1, _H1), lambda k: (0, 0)),
            pl.BlockSpec((_H1, _E), lambda k: (0, 0)),
            pl.BlockSpec((1, _E), lambda k: (0, 0)),
        ],
        out_specs=[
            pl.BlockSpec((_B, 2), lambda k: (0, 0)),
            pl.BlockSpec((_B, 2), lambda k: (0, 0)),
        ],
        out_shape=[
            jax.ShapeDtypeStruct((_B, 2), jnp.float32),
            jax.ShapeDtypeStruct((_B, 2), jnp.int32),
        ],
        scratch_shapes=[pltpu.VMEM((8, _D), jnp.float32)],
        compiler_params=pltpu.CompilerParams(
            dimension_semantics=("arbitrary",)),
        interpret=interpret,
    )(x2, W1, b1.reshape(1, _H1), W2, b2.reshape(1, _E))
    return w, idx


def kernel(x, W1, b1, W2, b2):
    return _run(x, W1, b1, W2, b2)


# speedup vs baseline: 2.3915x; 1.0039x over previous
import functools

import jax
import jax.numpy as jnp
from jax import lax
from jax.experimental import pallas as pl
from jax.experimental.pallas import tpu as pltpu

_B, _T, _D = 4, 8192, 2048
_H1, _E = 256, 64
_ROWS = 1024
_K = (_B * _T) // _ROWS
_KPB = _T // _ROWS


def _body(x_ref, w1_ref, b1_ref, w2_ref, b2_ref, w_ref, i_ref, acc_ref):
    k = pl.program_id(0)
    b = k // _KPB

    row = lax.broadcasted_iota(jnp.int32, (8, 1), 0)
    psum = jnp.sum(x_ref[...], axis=0, keepdims=True)
    part = jnp.where(row % _B == b, psum, 0.0)

    @pl.when(k == 0)
    def _init():
        acc_ref[...] = part

    @pl.when(k > 0)
    def _acc():
        acc_ref[...] += part

    @pl.when(k == _K - 1)
    def _gate():
        xm = acc_ref[0:_B, :] * (1.0 / _T)
        h = jnp.maximum(
            jnp.dot(xm, w1_ref[...], preferred_element_type=jnp.float32)
            + b1_ref[...], 0.0)
        g = (jnp.dot(h, w2_ref[...], preferred_element_type=jnp.float32)
             + b2_ref[...])
        gmax = jnp.max(g, axis=-1, keepdims=True)
        e = jnp.exp(g - gmax)
        p = e / jnp.sum(e, axis=-1, keepdims=True)
        iota = lax.broadcasted_iota(jnp.int32, p.shape, 1)
        v1 = jnp.max(p, axis=-1, keepdims=True)
        i1 = jnp.min(jnp.where(p == v1, iota, _E), axis=-1, keepdims=True)
        p2 = jnp.where(iota == i1, -jnp.inf, p)
        v2 = jnp.max(p2, axis=-1, keepdims=True)
        i2 = jnp.min(jnp.where(p2 == v2, iota, _E), axis=-1, keepdims=True)
        s = v1 + v2
        w_ref[...] = jnp.concatenate([v1 / s, v2 / s], axis=1)
        i_ref[...] = jnp.concatenate([i1, i2], axis=1)


@functools.partial(jax.jit, static_argnames=("interpret",))
def _run(x, W1, b1, W2, b2, interpret=False):
    x2 = x.reshape(_B * _T, _D)
    w, idx = pl.pallas_call(
        _body,
        grid=(_K,),
        in_specs=[
            pl.BlockSpec((_ROWS, _D), lambda k: (k, 0)),
            pl.BlockSpec((_D, _H1), lambda k: (0, 0)),
            pl.BlockSpec((1, _H1), lambda k: (0, 0)),
            pl.BlockSpec((_H1, _E), lambda k: (0, 0)),
            pl.BlockSpec((1, _E), lambda k: (0, 0)),
        ],
        out_specs=[
            pl.BlockSpec((_B, 2), lambda k: (0, 0)),
            pl.BlockSpec((_B, 2), lambda k: (0, 0)),
        ],
        out_shape=[
            jax.ShapeDtypeStruct((_B, 2), jnp.float32),
            jax.ShapeDtypeStruct((_B, 2), jnp.int32),
        ],
        scratch_shapes=[pltpu.VMEM((8, _D), jnp.float32)],
        compiler_params=pltpu.CompilerParams(
            dimension_semantics=("arbitrary",)),
        interpret=interpret,
    )(x2, W1, b1.reshape(1, _H1), W2, b2.reshape(1, _E))
    return w, idx


def kernel(x, W1, b1, W2, b2):
    return _run(x, W1, b1, W2, b2)
